# Initial kernel scaffold; baseline (speedup 1.0000x reference)
import jax
import jax.numpy as jnp
from jax.experimental import pallas as pl


def kernel(enc_input, table):
    raise NotImplementedError("write your pallas kernel here")


# speedup vs baseline: 4.8455x; 4.8455x over previous
import functools

import jax
import jax.numpy as jnp
from jax import lax
from jax.experimental import pallas as pl
from jax.experimental.pallas import tpu as pltpu
from jax.experimental.pallas import tpu_sc as plsc

BATCH = 1024
SEQ = 200
D = 128
N = BATCH * SEQ

CHUNK = 128


@functools.lru_cache(maxsize=None)
def _build(n_rows, d):
    info = plsc.get_sparse_core_info()
    nc, ns = info.num_cores, info.num_subcores
    nw = nc * ns
    per_w = n_rows // nw
    n_chunks = per_w // CHUNK

    mesh = plsc.VectorSubcoreMesh(core_axis_name="c", subcore_axis_name="s")

    @functools.partial(
        pl.kernel,
        mesh=mesh,
        out_type=jax.ShapeDtypeStruct((n_rows, d), jnp.float32),
        scratch_types=[
            pltpu.VMEM((CHUNK,), jnp.int32),
            pltpu.VMEM((CHUNK, d), jnp.float32),
            pltpu.SemaphoreType.DMA,
        ],
    )
    def gather_kernel(idx_hbm, table_hbm, out_hbm, idx_v, rows_v, sem):
        wid = lax.axis_index("s") * nc + lax.axis_index("c")
        base = wid * per_w

        def body(g, carry):
            off = base + g * CHUNK
            pltpu.sync_copy(idx_hbm.at[pl.ds(off, CHUNK)], idx_v)
            pltpu.async_copy(table_hbm.at[idx_v], rows_v, sem).wait()
            pltpu.sync_copy(rows_v, out_hbm.at[pl.ds(off, CHUNK)])
            return carry

        lax.fori_loop(0, n_chunks, body, 0)

    return gather_kernel


def kernel(enc_input, table):
    idx = enc_input.reshape(-1).astype(jnp.int32)
    out = _build(N, D)(idx, table)
    return out.reshape(BATCH, SEQ, D)


# speedup vs baseline: 6.5924x; 1.3605x over previous
import functools

import jax
import jax.numpy as jnp
from jax import lax
from jax.experimental import pallas as pl
from jax.experimental.pallas import tpu as pltpu
from jax.experimental.pallas import tpu_sc as plsc

BATCH = 1024
SEQ = 200
D = 128
N = BATCH * SEQ

CHUNK = 128


@functools.lru_cache(maxsize=None)
def _build(n_rows, d):
    info = plsc.get_sparse_core_info()
    nc, ns = info.num_cores, info.num_subcores
    nw = nc * ns
    per_w = n_rows // nw
    n_chunks = per_w // CHUNK

    mesh = plsc.VectorSubcoreMesh(core_axis_name="c", subcore_axis_name="s")

    @functools.partial(
        pl.kernel,
        mesh=mesh,
        out_type=jax.ShapeDtypeStruct((n_rows, d), jnp.float32),
        scratch_types=[
            pltpu.VMEM((n_chunks, CHUNK), jnp.int32),
            pltpu.VMEM((CHUNK, d), jnp.float32),
            pltpu.VMEM((CHUNK, d), jnp.float32),
            pltpu.SemaphoreType.DMA,
            pltpu.SemaphoreType.DMA,
            pltpu.SemaphoreType.DMA,
        ],
    )
    def gather_kernel(idx_hbm, table_hbm, out_hbm, idx_v, rows0, rows1,
                      gsem, ssem0, ssem1):
        wid = lax.axis_index("s") * nc + lax.axis_index("c")
        base = wid * per_w
        rows = (rows0, rows1)
        ssems = (ssem0, ssem1)

        pltpu.sync_copy(idx_hbm.at[wid], idx_v)

        for b in range(2):
            pltpu.async_copy(table_hbm.at[idx_v.at[b]], rows[b], gsem).wait()
            pltpu.async_copy(
                rows[b], out_hbm.at[pl.ds(base + b * CHUNK, CHUNK)], ssems[b])

        def outer(k, carry):
            gg = 2 * k
            for b in range(2):
                g = gg + b
                off = base + g * CHUNK
                dst = out_hbm.at[pl.ds(off, CHUNK)]
                pltpu.make_async_copy(rows[b], dst, ssems[b]).wait()
                pltpu.async_copy(table_hbm.at[idx_v.at[g]], rows[b], gsem).wait()
                pltpu.async_copy(rows[b], dst, ssems[b])
            return carry

        lax.fori_loop(1, n_chunks // 2, outer, 0)

        for b in range(2):
            off = base + (n_chunks - 2 + b) * CHUNK
            pltpu.make_async_copy(
                rows[b], out_hbm.at[pl.ds(off, CHUNK)], ssems[b]).wait()

    return gather_kernel


def kernel(enc_input, table):
    nw = 32
    idx = enc_input.reshape(nw, -1, CHUNK).astype(jnp.int32)
    out = _build(N, D)(idx, table)
    return out.reshape(BATCH, SEQ, D)


# speedup vs baseline: 7.9994x; 1.2134x over previous
import functools

import jax
import jax.numpy as jnp
from jax import lax
from jax.experimental import pallas as pl
from jax.experimental.pallas import tpu as pltpu
from jax.experimental.pallas import tpu_sc as plsc

BATCH = 1024
SEQ = 200
D = 128
N = BATCH * SEQ

CHUNK = 128
NBUF = 4


@functools.lru_cache(maxsize=None)
def _build(n_rows, d):
    info = plsc.get_sparse_core_info()
    nc, ns = info.num_cores, info.num_subcores
    nw = nc * ns
    per_w = n_rows // nw
    n_chunks = per_w // CHUNK

    mesh = plsc.VectorSubcoreMesh(core_axis_name="c", subcore_axis_name="s")

    @functools.partial(
        pl.kernel,
        mesh=mesh,
        out_type=jax.ShapeDtypeStruct((n_rows, d), jnp.float32),
        scratch_types=[
            pltpu.VMEM((n_chunks, CHUNK), jnp.int32),
        ] + [pltpu.VMEM((CHUNK, d), jnp.float32)] * NBUF
          + [pltpu.SemaphoreType.DMA] * (2 * NBUF),
    )
    def gather_kernel(idx_hbm, table_hbm, out_hbm, idx_v, *bufs_and_sems):
        rows = bufs_and_sems[:NBUF]
        gsems = bufs_and_sems[NBUF:2 * NBUF]
        ssems = bufs_and_sems[2 * NBUF:]

        wid = lax.axis_index("s") * nc + lax.axis_index("c")
        base = wid * per_w

        def gstart(g, b):
            pltpu.async_copy(table_hbm.at[idx_v.at[g]], rows[b], gsems[b])

        def gwait(g, b):
            pltpu.make_async_copy(
                table_hbm.at[idx_v.at[g]], rows[b], gsems[b]).wait()

        def sstart(g, b):
            pltpu.async_copy(
                rows[b], out_hbm.at[pl.ds(base + g * CHUNK, CHUNK)], ssems[b])

        def swait(g, b):
            pltpu.make_async_copy(
                rows[b], out_hbm.at[pl.ds(base + g * CHUNK, CHUNK)],
                ssems[b]).wait()

        pltpu.sync_copy(idx_hbm.at[wid], idx_v)

        gstart(0, 0)
        gstart(1, 1)

        for g in range(2):
            gwait(g, g)
            sstart(g, g)
            gstart(g + 2, g + 2)
        for g in range(2, 4):
            gwait(g, g)
            sstart(g, g)
            swait(g - 2, g - 2)
            gstart(g + 2, g - 2)

        def outer(k, carry):
            gg = NBUF * k
            for j in range(NBUF):
                g = gg + j
                b = j
                nb = (j + 2) % NBUF
                gwait(g, b)
                sstart(g, b)
                swait(g - 2, nb)
                gstart(g + 2, nb)
            return carry

        lax.fori_loop(1, (n_chunks - 2) // NBUF, outer, 0)

        for g in range(n_chunks - 2, n_chunks):
            b = g % NBUF
            gwait(g, b)
            sstart(g, b)

        for g in range(n_chunks - NBUF, n_chunks):
            swait(g, g % NBUF)

    return gather_kernel


def kernel(enc_input, table):
    nw = 32
    idx = enc_input.reshape(nw, -1, CHUNK).astype(jnp.int32)
    out = _build(N, D)(idx, table)
    return out.reshape(BATCH, SEQ, D)


# speedup vs baseline: 8.0249x; 1.0032x over previous
import functools

import jax
import jax.numpy as jnp
from jax import lax
from jax.experimental import pallas as pl
from jax.experimental.pallas import tpu as pltpu
from jax.experimental.pallas import tpu_sc as plsc

BATCH = 1024
SEQ = 200
D = 128
N = BATCH * SEQ

CHUNK = 128
NBUF = 5
DEPTH = 3


@functools.lru_cache(maxsize=None)
def _build(n_rows, d):
    info = plsc.get_sparse_core_info()
    nc, ns = info.num_cores, info.num_subcores
    nw = nc * ns
    per_w = n_rows // nw
    n_chunks = per_w // CHUNK
    assert n_chunks % NBUF == 0 and DEPTH < NBUF

    mesh = plsc.VectorSubcoreMesh(core_axis_name="c", subcore_axis_name="s")

    @functools.partial(
        pl.kernel,
        mesh=mesh,
        out_type=jax.ShapeDtypeStruct((n_rows, d), jnp.float32),
        scratch_types=[
            pltpu.VMEM((n_chunks, CHUNK), jnp.int32),
        ] + [pltpu.VMEM((CHUNK, d), jnp.float32)] * NBUF
          + [pltpu.SemaphoreType.DMA] * (2 * NBUF),
    )
    def gather_kernel(idx_hbm, table_hbm, out_hbm, idx_v, *bufs_and_sems):
        rows = bufs_and_sems[:NBUF]
        gsems = bufs_and_sems[NBUF:2 * NBUF]
        ssems = bufs_and_sems[2 * NBUF:]

        wid = lax.axis_index("s") * nc + lax.axis_index("c")
        base = wid * per_w

        def gstart(g, b):
            pltpu.async_copy(table_hbm.at[idx_v.at[g]], rows[b], gsems[b])

        def gwait(g, b):
            pltpu.make_async_copy(
                table_hbm.at[idx_v.at[g]], rows[b], gsems[b]).wait()

        def sstart(g, b):
            pltpu.async_copy(
                rows[b], out_hbm.at[pl.ds(base + g * CHUNK, CHUNK)], ssems[b])

        def swait(g, b):
            pltpu.make_async_copy(
                rows[b], out_hbm.at[pl.ds(base + g * CHUNK, CHUNK)],
                ssems[b]).wait()

        pltpu.sync_copy(idx_hbm.at[wid], idx_v)

        for g in range(DEPTH):
            gstart(g, g % NBUF)

        for g in range(NBUF):
            gwait(g, g)
            sstart(g, g)
            gi = g + DEPTH
            if gi < n_chunks:
                if gi - NBUF >= 0:
                    swait(gi - NBUF, gi % NBUF)
                gstart(gi, gi % NBUF)

        def outer(k, carry):
            gg = NBUF * k
            for j in range(NBUF):
                g = gg + j
                b = j
                nb = (j + DEPTH) % NBUF
                gwait(g, b)
                sstart(g, b)
                swait(g + DEPTH - NBUF, nb)

                @pl.when(g + DEPTH < n_chunks)
                def _():
                    gstart(g + DEPTH, nb)
            return carry

        lax.fori_loop(1, n_chunks // NBUF, outer, 0)

        for g in range(n_chunks + DEPTH - NBUF, n_chunks):
            swait(g, g % NBUF)

    return gather_kernel


def kernel(enc_input, table):
    nw = 32
    idx = enc_input.reshape(nw, -1, CHUNK).astype(jnp.int32)
    out = _build(N, D)(idx, table)
    return out.reshape(BATCH, SEQ, D)
